# baseline (speedup 1.0000x reference)
import functools

import jax
import jax.numpy as jnp
import numpy as np
from jax import lax
from jax.experimental import pallas as pl
from jax.experimental.pallas import tpu as pltpu
from jax.experimental.pallas import tpu_sc as plsc

B, T, D = 4, 4096, 2048
K = 4
C = 2048
NUM_TOKENS = 8192
HALF = 1024
MODEL_DIM = 2048

ITEMS = B * T
NB = 16
ROWS = NB * K
HALF_W = HALF // 2
LANES = 16

NC = 2
NS = 16

_HI_MASK = np.int32(-65536)
_LO_MASK = np.int32(0x0000FFFF)
_RND = np.int32(0x8000)


_RB = 2048


def _pack_body(x_ref, out_ref):
    x = x_ref[...]
    a = x[:, :HALF_W].astype(jnp.bfloat16)
    b = x[:, HALF_W:].astype(jnp.bfloat16)
    ai = lax.bitcast_convert_type(a, jnp.int16).astype(jnp.int32)
    bi = lax.bitcast_convert_type(b, jnp.int16).astype(jnp.int32)
    out_ref[...] = lax.bitwise_or(
        lax.shift_left(bi, 16), lax.bitwise_and(ai, _LO_MASK)
    )


def _tc_pack_table(table):
    return pl.pallas_call(
        _pack_body,
        grid=(NUM_TOKENS // _RB,),
        in_specs=[pl.BlockSpec((_RB, HALF), lambda r: (r, 0))],
        out_specs=pl.BlockSpec((_RB, HALF_W), lambda r: (r, 0)),
        out_shape=jax.ShapeDtypeStruct((NUM_TOKENS, HALF_W), jnp.int32),
    )(table)


def _sc_gather_sum():
    nw = NC * NS
    ipw = ITEMS // nw
    nblk = ipw // NB
    mesh = plsc.VectorSubcoreMesh(
        core_axis_name="c", subcore_axis_name="s", num_cores=NC, num_subcores=NS
    )

    wpb = T // (ITEMS // nw)

    @functools.partial(
        pl.kernel,
        out_type=jax.ShapeDtypeStruct((ITEMS, HALF_W), jnp.int32),
        mesh=mesh,
        cost_estimate=pl.CostEstimate(
            flops=ITEMS * K * HALF * 2,
            bytes_accessed=ITEMS * K * HALF * 2 + ITEMS * HALF * 2,
            transcendentals=0,
        ),
        scratch_types=[
            pltpu.VMEM((K, ITEMS // (NC * NS)), jnp.int32),
            pltpu.VMEM((ipw * K,), jnp.int32),
            pltpu.VMEM((2, ROWS, HALF_W), jnp.int32),
            pltpu.VMEM((2, NB, HALF_W), jnp.int32),
            pltpu.SemaphoreType.DMA,
            pltpu.SemaphoreType.DMA,
        ],
    )
    def sc_kernel(idx_hbm, table_hbm, out_hbm, idx_raw, idx_v, rows_v, out_v, gsem, osem):
        wid = lax.axis_index("s") * NC + lax.axis_index("c")
        base_item = wid * ipw

        b = wid // wpb
        t0 = (wid % wpb) * ipw
        pltpu.sync_copy(idx_hbm.at[pl.ds(b * K, K), pl.ds(t0, ipw)], idx_raw)
        for k in range(K):
            def prep_body(blk, _, k=k):
                v = idx_raw[k, pl.ds(blk * NB, NB)] + np.int32(k * C)
                idx_v[pl.ds(blk * (K * NB) + k * NB, NB)] = v
                return 0
            lax.fori_loop(0, nblk, prep_body, 0)

        def start_gather(blk, buf):
            pltpu.async_copy(
                table_hbm.at[idx_v.at[pl.ds(blk * ROWS, ROWS)]],
                rows_v.at[buf],
                gsem,
            )

        def wait_gather(buf):
            pltpu.make_async_copy(
                table_hbm.at[idx_v.at[pl.ds(0, ROWS)]], rows_v.at[buf], gsem
            ).wait()

        def wait_out(par):
            pltpu.make_async_copy(
                out_v.at[par], out_hbm.at[pl.ds(0, NB)], osem
            ).wait()

        def lo_f32(x):
            return lax.bitcast_convert_type(lax.shift_left(x, 16), jnp.float32)

        def hi_f32(x):
            return lax.bitcast_convert_type(lax.bitwise_and(x, _HI_MASK), jnp.float32)

        start_gather(0, 0)

        def blk_pair(h, _):
            for par in (0, 1):
                blk = 2 * h + par
                nxt = jnp.minimum(blk + 1, nblk - 1)
                start_gather(nxt, 1 - par)
                wait_gather(par)

                @pl.when(h > 0)
                def _():
                    wait_out(par)

                @plsc.parallel_loop(0, HALF_W // LANES, 1, unroll=2)
                def _(j):
                    s = pl.ds(j * LANES, LANES)
                    for i in range(NB):
                        r = [rows_v[par, m * NB + i, s] for m in range(K)]
                        s_lo = (lo_f32(r[0]) + lo_f32(r[1])) + (lo_f32(r[2]) + lo_f32(r[3]))
                        s_hi = (hi_f32(r[0]) + hi_f32(r[1])) + (hi_f32(r[2]) + hi_f32(r[3]))
                        lo_w = lax.shift_right_logical(
                            lax.bitcast_convert_type(s_lo, jnp.int32), 16
                        )
                        hi_w = lax.bitwise_and(
                            lax.bitcast_convert_type(s_hi, jnp.int32), _HI_MASK
                        )
                        out_v[par, i, s] = lax.bitwise_or(hi_w, lo_w)

                pltpu.async_copy(
                    out_v.at[par],
                    out_hbm.at[pl.ds(base_item + blk * NB, NB)],
                    osem,
                )
            return 0

        lax.fori_loop(0, nblk // 2, blk_pair, 0)
        wait_gather(0)
        wait_out(0)
        wait_out(1)

    return sc_kernel


_TB = 512


def _fc_body(inst_ids, x_ref, w_ref, b_ref, inst_ref, out_ref):
    x = x_ref[0].astype(jnp.bfloat16)
    w = w_ref[...]
    acc = lax.dot_general(
        x, w, (((1,), (1,)), ((), ())),
        preferred_element_type=jnp.float32,
    )
    out_ref[0] = acc + b_ref[...] + inst_ref[0]


def _tc_fc(x, w, b, inst_ids, inst_table):
    grid_spec = pltpu.PrefetchScalarGridSpec(
        num_scalar_prefetch=1,
        grid=(B, T // _TB),
        in_specs=[
            pl.BlockSpec((1, _TB, D), lambda bi, ti, ids: (bi, ti, 0)),
            pl.BlockSpec((HALF, D), lambda bi, ti, ids: (0, 0)),
            pl.BlockSpec((1, HALF), lambda bi, ti, ids: (0, 0)),
            pl.BlockSpec((1, 1, HALF), lambda bi, ti, ids: (ids[bi], 0, 0)),
        ],
        out_specs=pl.BlockSpec((1, _TB, HALF), lambda bi, ti, ids: (bi, ti, 0)),
    )
    return pl.pallas_call(
        _fc_body,
        grid_spec=grid_spec,
        out_shape=jax.ShapeDtypeStruct((B, T, MODEL_DIM), jnp.float32),
    )(inst_ids, x, w.astype(jnp.bfloat16), b.reshape(1, HALF),
      inst_table.reshape(-1, 1, HALF))


def _combine_body(inst_ids, buf_ref, tok_ref, inst_ref, out_ref):
    tok = tok_ref[0]
    inst = inst_ref[0]
    lo = lax.bitcast_convert_type(lax.shift_left(tok, 16), jnp.float32)
    hi = lax.bitcast_convert_type(lax.bitwise_and(tok, _HI_MASK), jnp.float32)
    out_ref[0, :, :HALF_W] = lo + inst[:, :HALF_W]
    out_ref[0, :, HALF_W:] = hi + inst[:, HALF_W:]


def _tc_combine(buf, tok_sum_w, inst_ids, inst_table):
    grid_spec = pltpu.PrefetchScalarGridSpec(
        num_scalar_prefetch=1,
        grid=(B, T // _TB),
        in_specs=[
            pl.BlockSpec(memory_space=pl.ANY),
            pl.BlockSpec((1, _TB, HALF_W), lambda bi, ti, ids: (bi, ti, 0)),
            pl.BlockSpec((1, 1, HALF), lambda bi, ti, ids: (ids[bi], 0, 0)),
        ],
        out_specs=pl.BlockSpec((1, _TB, HALF), lambda bi, ti, ids: (bi, ti, 1)),
    )
    return pl.pallas_call(
        _combine_body,
        grid_spec=grid_spec,
        out_shape=jax.ShapeDtypeStruct((B, T, MODEL_DIM), jnp.float32),
        input_output_aliases={1: 0},
    )(inst_ids, buf, tok_sum_w, inst_table.reshape(-1, 1, HALF))


def kernel(input, target_masked, target_inst_id, W_fc, b_fc, target_table, inst_table):
    tok_2d = target_masked.reshape(B * K, T)

    table_w = _tc_pack_table(target_table)
    tok_sum_w = _sc_gather_sum()(tok_2d, table_w)
    tok_sum_w = tok_sum_w.reshape(B, T, HALF_W)

    buf = _tc_fc(input, W_fc, b_fc, target_inst_id, inst_table)
    return _tc_combine(buf, tok_sum_w, target_inst_id, inst_table)


# speedup vs baseline: 1.0322x; 1.0322x over previous
import functools

import jax
import jax.numpy as jnp
import numpy as np
from jax import lax
from jax.experimental import pallas as pl
from jax.experimental.pallas import tpu as pltpu
from jax.experimental.pallas import tpu_sc as plsc

B, T, D = 4, 4096, 2048
K = 4
C = 2048
NUM_TOKENS = 8192
HALF = 1024
MODEL_DIM = 2048

ITEMS = B * T
NB = 16
ROWS = NB * K
HALF_W = HALF // 2
LANES = 16

NC = 2
NS = 16

_HI_MASK = np.int32(-65536)
_LO_MASK = np.int32(0x0000FFFF)
_RND = np.int32(0x8000)


_RB = 2048


def _pack_body(x_ref, out_ref):
    x = x_ref[...]
    a = x[:, :HALF_W].astype(jnp.bfloat16)
    b = x[:, HALF_W:].astype(jnp.bfloat16)
    ai = lax.bitcast_convert_type(a, jnp.int16).astype(jnp.int32)
    bi = lax.bitcast_convert_type(b, jnp.int16).astype(jnp.int32)
    out_ref[...] = lax.bitwise_or(
        lax.shift_left(bi, 16), lax.bitwise_and(ai, _LO_MASK)
    )


def _tc_pack_table(table):
    return pl.pallas_call(
        _pack_body,
        grid=(NUM_TOKENS // _RB,),
        in_specs=[pl.BlockSpec((_RB, HALF), lambda r: (r, 0))],
        out_specs=pl.BlockSpec((_RB, HALF_W), lambda r: (r, 0)),
        out_shape=jax.ShapeDtypeStruct((NUM_TOKENS, HALF_W), jnp.int32),
    )(table)


def _sc_gather_sum():
    nw = NC * NS
    ipw = ITEMS // nw
    nblk = ipw // NB
    mesh = plsc.VectorSubcoreMesh(
        core_axis_name="c", subcore_axis_name="s", num_cores=NC, num_subcores=NS
    )

    wpb = T // (ITEMS // nw)

    @functools.partial(
        pl.kernel,
        out_type=jax.ShapeDtypeStruct((ITEMS, HALF_W), jnp.int32),
        mesh=mesh,
        cost_estimate=pl.CostEstimate(
            flops=ITEMS * K * HALF * 2,
            bytes_accessed=ITEMS * K * HALF * 2 + ITEMS * HALF * 2,
            transcendentals=0,
        ),
        scratch_types=[
            pltpu.VMEM((K, ITEMS // (NC * NS)), jnp.int32),
            pltpu.VMEM((ipw * K,), jnp.int32),
            pltpu.VMEM((2, ROWS, HALF_W), jnp.int32),
            pltpu.VMEM((2, NB, HALF_W), jnp.int32),
            pltpu.SemaphoreType.DMA,
            pltpu.SemaphoreType.DMA,
        ],
    )
    def sc_kernel(idx_hbm, table_hbm, out_hbm, idx_raw, idx_v, rows_v, out_v, gsem, osem):
        wid = lax.axis_index("s") * NC + lax.axis_index("c")
        base_item = wid * ipw

        b = wid // wpb
        t0 = (wid % wpb) * ipw
        pltpu.sync_copy(idx_hbm.at[pl.ds(b * K, K), pl.ds(t0, ipw)], idx_raw)
        for k in range(K):
            def prep_body(blk, _, k=k):
                v = idx_raw[k, pl.ds(blk * NB, NB)] + np.int32(k * C)
                idx_v[pl.ds(blk * (K * NB) + k * NB, NB)] = v
                return 0
            lax.fori_loop(0, nblk, prep_body, 0)

        def start_gather(blk, buf):
            pltpu.async_copy(
                table_hbm.at[idx_v.at[pl.ds(blk * ROWS, ROWS)]],
                rows_v.at[buf],
                gsem,
            )

        def wait_gather(buf):
            pltpu.make_async_copy(
                table_hbm.at[idx_v.at[pl.ds(0, ROWS)]], rows_v.at[buf], gsem
            ).wait()

        def wait_out(par):
            pltpu.make_async_copy(
                out_v.at[par], out_hbm.at[pl.ds(0, NB)], osem
            ).wait()

        def lo_f32(x):
            return lax.bitcast_convert_type(lax.shift_left(x, 16), jnp.float32)

        def hi_f32(x):
            return lax.bitcast_convert_type(lax.bitwise_and(x, _HI_MASK), jnp.float32)

        start_gather(0, 0)

        def blk_pair(h, _):
            for par in (0, 1):
                blk = 2 * h + par
                nxt = jnp.minimum(blk + 1, nblk - 1)
                start_gather(nxt, 1 - par)
                wait_gather(par)

                @pl.when(h > 0)
                def _():
                    wait_out(par)

                @plsc.parallel_loop(0, HALF_W // LANES, 1, unroll=2)
                def _(j):
                    s = pl.ds(j * LANES, LANES)
                    for i in range(NB):
                        r = [rows_v[par, m * NB + i, s] for m in range(K)]
                        s_lo = (lo_f32(r[0]) + lo_f32(r[1])) + (lo_f32(r[2]) + lo_f32(r[3]))
                        s_hi = (hi_f32(r[0]) + hi_f32(r[1])) + (hi_f32(r[2]) + hi_f32(r[3]))
                        lo_w = lax.shift_right_logical(
                            lax.bitcast_convert_type(s_lo, jnp.int32), 16
                        )
                        hi_w = lax.bitwise_and(
                            lax.bitcast_convert_type(s_hi, jnp.int32), _HI_MASK
                        )
                        out_v[par, i, s] = lax.bitwise_or(hi_w, lo_w)

                pltpu.async_copy(
                    out_v.at[par],
                    out_hbm.at[pl.ds(base_item + blk * NB, NB)],
                    osem,
                )
            return 0

        lax.fori_loop(0, nblk // 2, blk_pair, 0)
        wait_gather(0)
        wait_out(0)
        wait_out(1)

    return sc_kernel


_TB = 1024


def _fc_body(inst_ids, x_ref, w_ref, b_ref, inst_ref, out_ref):
    x = x_ref[0].astype(jnp.bfloat16)
    w = w_ref[...]
    acc = lax.dot_general(
        x, w, (((1,), (1,)), ((), ())),
        preferred_element_type=jnp.float32,
    )
    out_ref[0] = acc + b_ref[...] + inst_ref[0]


def _tc_fc(x, w, b, inst_ids, inst_table):
    grid_spec = pltpu.PrefetchScalarGridSpec(
        num_scalar_prefetch=1,
        grid=(B, T // _TB),
        in_specs=[
            pl.BlockSpec((1, _TB, D), lambda bi, ti, ids: (bi, ti, 0)),
            pl.BlockSpec((HALF, D), lambda bi, ti, ids: (0, 0)),
            pl.BlockSpec((1, HALF), lambda bi, ti, ids: (0, 0)),
            pl.BlockSpec((1, 1, HALF), lambda bi, ti, ids: (ids[bi], 0, 0)),
        ],
        out_specs=pl.BlockSpec((1, _TB, HALF), lambda bi, ti, ids: (bi, ti, 0)),
    )
    return pl.pallas_call(
        _fc_body,
        grid_spec=grid_spec,
        out_shape=jax.ShapeDtypeStruct((B, T, MODEL_DIM), jnp.float32),
    )(inst_ids, x, w.astype(jnp.bfloat16), b.reshape(1, HALF),
      inst_table.reshape(-1, 1, HALF))


def _combine_body(inst_ids, buf_ref, tok_ref, inst_ref, out_ref):
    tok = tok_ref[0]
    inst = inst_ref[0]
    lo = lax.bitcast_convert_type(lax.shift_left(tok, 16), jnp.float32)
    hi = lax.bitcast_convert_type(lax.bitwise_and(tok, _HI_MASK), jnp.float32)
    out_ref[0, :, :HALF_W] = lo + inst[:, :HALF_W]
    out_ref[0, :, HALF_W:] = hi + inst[:, HALF_W:]


def _tc_combine(buf, tok_sum_w, inst_ids, inst_table):
    grid_spec = pltpu.PrefetchScalarGridSpec(
        num_scalar_prefetch=1,
        grid=(B, T // _TB),
        in_specs=[
            pl.BlockSpec(memory_space=pl.ANY),
            pl.BlockSpec((1, _TB, HALF_W), lambda bi, ti, ids: (bi, ti, 0)),
            pl.BlockSpec((1, 1, HALF), lambda bi, ti, ids: (ids[bi], 0, 0)),
        ],
        out_specs=pl.BlockSpec((1, _TB, HALF), lambda bi, ti, ids: (bi, ti, 1)),
    )
    return pl.pallas_call(
        _combine_body,
        grid_spec=grid_spec,
        out_shape=jax.ShapeDtypeStruct((B, T, MODEL_DIM), jnp.float32),
        input_output_aliases={1: 0},
    )(inst_ids, buf, tok_sum_w, inst_table.reshape(-1, 1, HALF))


def kernel(input, target_masked, target_inst_id, W_fc, b_fc, target_table, inst_table):
    tok_2d = target_masked.reshape(B * K, T)

    table_w = _tc_pack_table(target_table)
    tok_sum_w = _sc_gather_sum()(tok_2d, table_w)
    tok_sum_w = tok_sum_w.reshape(B, T, HALF_W)

    buf = _tc_fc(input, W_fc, b_fc, target_inst_id, inst_table)
    return _tc_combine(buf, tok_sum_w, target_inst_id, inst_table)
